# baseline (speedup 1.0000x reference)
import functools

import jax
import jax.numpy as jnp
from jax import lax
from jax.experimental import pallas as pl
from jax.experimental.pallas import tpu as pltpu
from jax.experimental.pallas import tpu_sc as plsc

_NUM_CLASSES = 80
_SHRINK = 0.2
_B = 8
_N = 100
_LEVELS = (
    (8, 64, 64, 6, 0),
    (16, 32, 32, 5, 4096),
    (32, 16, 16, 4, 5120),
    (64, 8, 8, 3, 5376),
    (128, 4, 4, 2, 5440),
)
_NPIX = 5456
_NPIXP = 5632
_F32 = jnp.float32
_I32 = jnp.int32


def _iota16():
    return lax.iota(_I32, 16)


def _csplat(c):
    return jnp.full((16,), c, dtype=_I32)


_CLS_CHUNK = 1024


def _process_level(level, b, y0, nrows, out_base, flush,
                   gt_hbm, msw_hbm, zeros_hbm, cls_hbm, regr_hbm,
                   boxes_v, msw_v, px1_v, py1_v, px2_v, py2_v, val_v,
                   bx1_v, by1_v, bx2_v, by2_v, besta, besti, rstage, cstage):
    stride, fh, fw, log2fw, p0_level = _LEVELS[level]
    npx = nrows * fw
    stride_f = _F32(stride)
    iot = _iota16()

    pltpu.sync_copy(gt_hbm.at[b], boxes_v)
    pltpu.sync_copy(msw_hbm.at[b], msw_v)

    for g in range(7):
        ridx = g * 16 + iot
        gmask = ridx < _N
        fidx = ridx * 5
        x1 = plsc.load_gather(boxes_v, [fidx], mask=gmask)
        y1 = plsc.load_gather(boxes_v, [fidx + 1], mask=gmask)
        x2 = plsc.load_gather(boxes_v, [fidx + 2], mask=gmask)
        y2 = plsc.load_gather(boxes_v, [fidx + 3], mask=gmask)
        valid = ((jnp.abs(x1) + jnp.abs(y1) + jnp.abs(x2) + jnp.abs(y2)) > 0.0) & gmask
        cx = (x1 + x2) * _F32(0.5)
        cy = (y1 + y2) * _F32(0.5)
        w = x2 - x1
        h = y2 - y1
        inv_s = _F32(1.0 / stride)
        sx1 = (cx - w * _F32(_SHRINK) * _F32(0.5)) * inv_s
        sy1 = (cy - h * _F32(_SHRINK) * _F32(0.5)) * inv_s
        sx2 = (cx + w * _F32(_SHRINK) * _F32(0.5)) * inv_s
        sy2 = (cy + h * _F32(_SHRINK) * _F32(0.5)) * inv_s
        p1 = jnp.clip(sx1.astype(_I32), 0, fw - 1)
        q1 = jnp.clip(sy1.astype(_I32), 0, fh - 1)
        c2 = sx2.astype(_I32)
        c2 = c2 + jnp.where(c2.astype(_F32) < sx2, 1, 0)
        r2 = sy2.astype(_I32)
        r2 = r2 + jnp.where(r2.astype(_F32) < sy2, 1, 0)
        p2 = jnp.clip(c2, p1 + 1, fw)
        q2 = jnp.clip(r2, q1 + 1, fh)
        sl = pl.ds(g * 16, 16)
        px1_v[sl] = p1
        py1_v[sl] = q1
        px2_v[sl] = p2
        py2_v[sl] = q2
        val_v[sl] = jnp.where(valid, 1, 0)
        bx1_v[sl] = x1
        by1_v[sl] = y1
        bx2_v[sl] = x2
        by2_v[sl] = y2

    big = jnp.full((16,), 1e7, dtype=_F32)
    zer = jnp.zeros((16,), dtype=_I32)

    def init_body(g, _):
        besta[pl.ds(g * 16, 16)] = big
        besti[pl.ds(g * 16, 16)] = zer
        return 0

    lax.fori_loop(0, npx // 16, init_body, 0)

    def box_grp_body(g, _):
        gs = pl.ds(g * 16, 16)
        p1v = px1_v[gs]
        p2v = px2_v[gs]
        q1v = py1_v[gs]
        q2v = py2_v[gs]
        vv = val_v[gs]
        x1v = bx1_v[gs]
        y1v = by1_v[gs]
        x2v = bx2_v[gs]
        y2v = by2_v[gs]
        for j in range(16):
            n = g * 16 + j
            v = vv[j]
            p1 = p1v[j]
            q1 = q1v[j]
            q2 = q2v[j]
            x1f = x1v[j]
            y1f = y1v[j]
            x2f = x2v[j]
            y2f = y2v[j]
            ry1 = jnp.maximum(q1, y0)
            ry2 = jnp.minimum(q2, y0 + nrows)
            ry2 = jnp.where(v > 0, jnp.maximum(ry2, ry1), ry1)
            rw = p2v[j] - p1
            m_in = iot < rw
            xsf = (p1 + iot).astype(_F32)
            sx = (xsf + _F32(0.5)) * stride_f
            dl = jnp.maximum(sx - x1f, 0.0)
            dr = jnp.maximum(x2f - sx, 0.0)
            dlr = dl + dr
            nvec = lax.broadcast(n, (16,))

            def row_body(y, _, dlr=dlr, m_in=m_in, y1f=y1f, y2f=y2f,
                         p1=p1, nvec=nvec):
                sy = (y.astype(_F32) + _F32(0.5)) * stride_f
                dt = jnp.maximum(sy - y1f, 0.0)
                db = jnp.maximum(y2f - sy, 0.0)
                area = dlr * (dt + db)
                loc = (y - y0) * fw + p1
                sl = pl.ds(loc, 16)
                cur = besta[sl]
                upd = m_in & (area < cur)
                besta[sl] = jnp.where(upd, area, cur)
                curi = besti[sl]
                besti[sl] = jnp.where(upd, nvec, curi)
                return 0

            lax.fori_loop(ry1, ry2, row_body, 0)
        return 0

    lax.fori_loop(0, 7, box_grp_body, 0)

    inv4s = _F32(1.0 / (4.0 * stride))
    p0 = p0_level + y0 * fw

    def make_grp_body(cls_off):
        def grp_body(g, _):
            base = g * 16
            lp = base + iot
            sl = pl.ds(base, 16)
            idxv = besti[sl]
            areav = besta[sl]
            pos = areav < 1e7
            posf = jnp.where(pos, _F32(1.0), _F32(0.0))
            x = lp & (fw - 1)
            y = y0 + lax.shift_right_logical(lp, log2fw)
            sx = (x.astype(_F32) + _F32(0.5)) * stride_f
            sy = (y.astype(_F32) + _F32(0.5)) * stride_f
            idx5 = idxv * 5
            bx1 = plsc.load_gather(boxes_v, [idx5])
            by1 = plsc.load_gather(boxes_v, [idx5 + 1])
            bx2 = plsc.load_gather(boxes_v, [idx5 + 2])
            by2 = plsc.load_gather(boxes_v, [idx5 + 3])
            labf = plsc.load_gather(boxes_v, [idx5 + 4])
            mw = plsc.load_gather(msw_v, [idx5 + level])
            dl = jnp.maximum(sx - bx1, 0.0)
            dt = jnp.maximum(sy - by1, 0.0)
            dr = jnp.maximum(bx2 - sx, 0.0)
            db = jnp.maximum(by2 - sy, 0.0)
            apn = jnp.minimum(dl, dr) * jnp.minimum(dt, db)
            apd = jnp.maximum(jnp.maximum(dl, dr) * jnp.maximum(dt, db),
                              1e-12)
            soft = jnp.where(pos, (apn / apd) * mw, _F32(1.0))
            lp6 = (out_base + lp) * 6
            plsc.store_scatter(rstage, [lp6], dl * inv4s * posf)
            plsc.store_scatter(rstage, [lp6 + 1], dt * inv4s * posf)
            plsc.store_scatter(rstage, [lp6 + 2], dr * inv4s * posf)
            plsc.store_scatter(rstage, [lp6 + 3], db * inv4s * posf)
            plsc.store_scatter(rstage, [lp6 + 4], soft)
            plsc.store_scatter(rstage, [lp6 + 5], posf)
            lp82 = (lp + cls_off) * 82
            labv = labf.astype(_I32)
            plsc.store_scatter(cstage, [lp82 + labv], posf, mask=pos)
            plsc.store_scatter(cstage, [lp82 + 80], soft)
            plsc.store_scatter(cstage, [lp82 + 81], posf)
            return 0

        return grp_body

    if flush:
        for c0 in range(0, npx, _CLS_CHUNK):
            cn = min(_CLS_CHUNK, npx - c0)
            pltpu.sync_copy(zeros_hbm.at[pl.ds(0, cn * 82)],
                            cstage.at[pl.ds(0, cn * 82)])
            lax.fori_loop(c0 // 16, (c0 + cn) // 16, make_grp_body(-c0), 0)
            pltpu.sync_copy(
                cstage.at[pl.ds(0, cn * 82)],
                cls_hbm.at[b, pl.ds((p0 + c0) * 82, cn * 82)])
        pltpu.sync_copy(rstage.at[pl.ds(0, npx * 6)],
                        regr_hbm.at[b, pl.ds(p0 * 6, npx * 6)])
    else:
        lax.fori_loop(0, npx // 16, make_grp_body(out_base), 0)


def _sc_body(gt_hbm, msw_hbm, zeros_hbm, cls_hbm, regr_hbm, boxes_v, msw_v,
             px1_v, py1_v, px2_v, py2_v, val_v,
             bx1_v, by1_v, bx2_v, by2_v, besta, besti, rstage, cstage):
    cid = lax.axis_index("c")
    sid = lax.axis_index("s")
    wid = sid * 2 + cid
    scr = (boxes_v, msw_v, px1_v, py1_v, px2_v, py2_v, val_v,
           bx1_v, by1_v, bx2_v, by2_v, besta, besti, rstage, cstage)
    hbm = (gt_hbm, msw_hbm, zeros_hbm, cls_hbm, regr_hbm)

    @pl.when(wid < 16)
    def _():
        b = lax.div(wid, 2)
        half = wid - b * 2
        _process_level(0, b, half * 32, 32, 0, True, *hbm, *scr)

    @pl.when((wid >= 16) & (wid < 24))
    def _():
        _process_level(1, wid - 16, wid * 0, 32, 0, True, *hbm, *scr)

    @pl.when(wid >= 24)
    def _():
        b = wid - 24
        z = b * 0
        pltpu.sync_copy(zeros_hbm.at[pl.ds(0, 512 * 82)],
                        cstage.at[pl.ds(0, 512 * 82)])
        _process_level(2, b, z, 16, 0, False, *hbm, *scr)
        _process_level(3, b, z, 8, 256, False, *hbm, *scr)
        _process_level(4, b, z, 4, 320, False, *hbm, *scr)
        pltpu.sync_copy(cstage.at[pl.ds(0, 512 * 82)],
                        cls_hbm.at[b, pl.ds(5120 * 82, 512 * 82)])
        pltpu.sync_copy(rstage.at[pl.ds(0, 512 * 6)],
                        regr_hbm.at[b, pl.ds(5120 * 6, 512 * 6)])


@jax.jit
def _sc_assign(gt_boxes, msw):
    mesh = plsc.VectorSubcoreMesh(core_axis_name="c", subcore_axis_name="s")
    f = pl.kernel(
        _sc_body,
        out_type=(
            jax.ShapeDtypeStruct((_B, _NPIXP * 82), _F32),
            jax.ShapeDtypeStruct((_B, _NPIXP * 6), _F32),
        ),
        mesh=mesh,
        compiler_params=pltpu.CompilerParams(needs_layout_passes=False),
        scratch_types=[
            pltpu.VMEM((_N * 5,), _F32),
            pltpu.VMEM((_N * 5,), _F32),
            pltpu.VMEM((112,), _I32),
            pltpu.VMEM((112,), _I32),
            pltpu.VMEM((112,), _I32),
            pltpu.VMEM((112,), _I32),
            pltpu.VMEM((112,), _I32),
            pltpu.VMEM((112,), _F32),
            pltpu.VMEM((112,), _F32),
            pltpu.VMEM((112,), _F32),
            pltpu.VMEM((112,), _F32),
            pltpu.VMEM((2064,), _F32),
            pltpu.VMEM((2064,), _I32),
            pltpu.VMEM((2048 * 6 + 16,), _F32),
            pltpu.VMEM((_CLS_CHUNK * 82,), _F32),
        ],
    )
    zeros = jnp.zeros((_CLS_CHUNK * 82,), _F32)
    return f(gt_boxes.reshape(_B, _N * 5), msw.reshape(_B, _N * 5), zeros)


def kernel(fm_shapes, gt_boxes, meta_select_weight):
    del fm_shapes
    cls_flat, regr_flat = _sc_assign(gt_boxes, meta_select_weight)
    regr_t = regr_flat[:, :_NPIX * 6].reshape(_B, _NPIX, 6)
    cls_t = cls_flat[:, :_NPIX * 82].reshape(_B, _NPIX, 82)
    return cls_t, regr_t


# speedup vs baseline: 1.1215x; 1.1215x over previous
import functools

import jax
import jax.numpy as jnp
from jax import lax
from jax.experimental import pallas as pl
from jax.experimental.pallas import tpu as pltpu
from jax.experimental.pallas import tpu_sc as plsc

_NUM_CLASSES = 80
_SHRINK = 0.2
_B = 8
_N = 100
_LEVELS = (
    (8, 64, 64, 6, 0),
    (16, 32, 32, 5, 4096),
    (32, 16, 16, 4, 5120),
    (64, 8, 8, 3, 5376),
    (128, 4, 4, 2, 5440),
)
_NPIX = 5456
_NPIXP = 5632
_F32 = jnp.float32
_I32 = jnp.int32


def _iota16():
    return lax.iota(_I32, 16)


def _csplat(c):
    return jnp.full((16,), c, dtype=_I32)


def _process_level(level, b, y0, nrows, out_base, flush,
                   gt_hbm, msw_hbm, regr_hbm, lab_hbm,
                   boxes_v, msw_v, px1_v, py1_v, px2_v, py2_v, val_v,
                   bx1_v, by1_v, bx2_v, by2_v, besta, besti, rstage, lstage):
    stride, fh, fw, log2fw, p0_level = _LEVELS[level]
    npx = nrows * fw
    stride_f = _F32(stride)
    iot = _iota16()

    pltpu.sync_copy(gt_hbm.at[b], boxes_v)
    pltpu.sync_copy(msw_hbm.at[b], msw_v)

    for g in range(7):
        ridx = g * 16 + iot
        gmask = ridx < _N
        fidx = ridx * 5
        x1 = plsc.load_gather(boxes_v, [fidx], mask=gmask)
        y1 = plsc.load_gather(boxes_v, [fidx + 1], mask=gmask)
        x2 = plsc.load_gather(boxes_v, [fidx + 2], mask=gmask)
        y2 = plsc.load_gather(boxes_v, [fidx + 3], mask=gmask)
        valid = ((jnp.abs(x1) + jnp.abs(y1) + jnp.abs(x2) + jnp.abs(y2)) > 0.0) & gmask
        cx = (x1 + x2) * _F32(0.5)
        cy = (y1 + y2) * _F32(0.5)
        w = x2 - x1
        h = y2 - y1
        inv_s = _F32(1.0 / stride)
        sx1 = (cx - w * _F32(_SHRINK) * _F32(0.5)) * inv_s
        sy1 = (cy - h * _F32(_SHRINK) * _F32(0.5)) * inv_s
        sx2 = (cx + w * _F32(_SHRINK) * _F32(0.5)) * inv_s
        sy2 = (cy + h * _F32(_SHRINK) * _F32(0.5)) * inv_s
        p1 = jnp.clip(sx1.astype(_I32), 0, fw - 1)
        q1 = jnp.clip(sy1.astype(_I32), 0, fh - 1)
        c2 = sx2.astype(_I32)
        c2 = c2 + jnp.where(c2.astype(_F32) < sx2, 1, 0)
        r2 = sy2.astype(_I32)
        r2 = r2 + jnp.where(r2.astype(_F32) < sy2, 1, 0)
        p2 = jnp.clip(c2, p1 + 1, fw)
        q2 = jnp.clip(r2, q1 + 1, fh)
        sl = pl.ds(g * 16, 16)
        px1_v[sl] = p1
        py1_v[sl] = q1
        px2_v[sl] = p2
        py2_v[sl] = q2
        val_v[sl] = jnp.where(valid, 1, 0)
        bx1_v[sl] = x1
        by1_v[sl] = y1
        bx2_v[sl] = x2
        by2_v[sl] = y2

    big = jnp.full((16,), 1e7, dtype=_F32)
    zer = jnp.zeros((16,), dtype=_I32)

    def init_body(g, _):
        besta[pl.ds(g * 16, 16)] = big
        besti[pl.ds(g * 16, 16)] = zer
        return 0

    lax.fori_loop(0, npx // 16, init_body, 0)

    def box_grp_body(g, _):
        gs = pl.ds(g * 16, 16)
        p1v = px1_v[gs]
        p2v = px2_v[gs]
        q1v = py1_v[gs]
        q2v = py2_v[gs]
        vv = val_v[gs]
        x1v = bx1_v[gs]
        y1v = by1_v[gs]
        x2v = bx2_v[gs]
        y2v = by2_v[gs]
        for j in range(16):
            n = g * 16 + j
            v = vv[j]
            p1 = p1v[j]
            q1 = q1v[j]
            q2 = q2v[j]
            x1f = x1v[j]
            y1f = y1v[j]
            x2f = x2v[j]
            y2f = y2v[j]
            ry1 = jnp.maximum(q1, y0)
            ry2 = jnp.minimum(q2, y0 + nrows)
            ry2 = jnp.where(v > 0, jnp.maximum(ry2, ry1), ry1)
            rw = p2v[j] - p1
            m_in = iot < rw
            xsf = (p1 + iot).astype(_F32)
            sx = (xsf + _F32(0.5)) * stride_f
            dl = jnp.maximum(sx - x1f, 0.0)
            dr = jnp.maximum(x2f - sx, 0.0)
            dlr = dl + dr
            nvec = lax.broadcast(n, (16,))

            def row_body(y, _, dlr=dlr, m_in=m_in, y1f=y1f, y2f=y2f,
                         p1=p1, nvec=nvec):
                sy = (y.astype(_F32) + _F32(0.5)) * stride_f
                dt = jnp.maximum(sy - y1f, 0.0)
                db = jnp.maximum(y2f - sy, 0.0)
                area = dlr * (dt + db)
                loc = (y - y0) * fw + p1
                sl = pl.ds(loc, 16)
                cur = besta[sl]
                upd = m_in & (area < cur)
                besta[sl] = jnp.where(upd, area, cur)
                curi = besti[sl]
                besti[sl] = jnp.where(upd, nvec, curi)
                return 0

            lax.fori_loop(ry1, ry2, row_body, 0)
        return 0

    lax.fori_loop(0, 7, box_grp_body, 0)

    inv4s = _F32(1.0 / (4.0 * stride))
    p0 = p0_level + y0 * fw

    def make_grp_body(cls_off):
        def grp_body(g, _):
            base = g * 16
            lp = base + iot
            sl = pl.ds(base, 16)
            idxv = besti[sl]
            areav = besta[sl]
            pos = areav < 1e7
            posf = jnp.where(pos, _F32(1.0), _F32(0.0))
            x = lp & (fw - 1)
            y = y0 + lax.shift_right_logical(lp, log2fw)
            sx = (x.astype(_F32) + _F32(0.5)) * stride_f
            sy = (y.astype(_F32) + _F32(0.5)) * stride_f
            idx5 = idxv * 5
            bx1 = plsc.load_gather(boxes_v, [idx5])
            by1 = plsc.load_gather(boxes_v, [idx5 + 1])
            bx2 = plsc.load_gather(boxes_v, [idx5 + 2])
            by2 = plsc.load_gather(boxes_v, [idx5 + 3])
            labf = plsc.load_gather(boxes_v, [idx5 + 4])
            mw = plsc.load_gather(msw_v, [idx5 + level])
            dl = jnp.maximum(sx - bx1, 0.0)
            dt = jnp.maximum(sy - by1, 0.0)
            dr = jnp.maximum(bx2 - sx, 0.0)
            db = jnp.maximum(by2 - sy, 0.0)
            apn = jnp.minimum(dl, dr) * jnp.minimum(dt, db)
            apd = jnp.maximum(jnp.maximum(dl, dr) * jnp.maximum(dt, db),
                              1e-12)
            soft = jnp.where(pos, (apn / apd) * mw, _F32(1.0))
            lp6 = (out_base + lp) * 6
            plsc.store_scatter(rstage, [lp6], dl * inv4s * posf)
            plsc.store_scatter(rstage, [lp6 + 1], dt * inv4s * posf)
            plsc.store_scatter(rstage, [lp6 + 2], dr * inv4s * posf)
            plsc.store_scatter(rstage, [lp6 + 3], db * inv4s * posf)
            plsc.store_scatter(rstage, [lp6 + 4], soft)
            plsc.store_scatter(rstage, [lp6 + 5], posf)
            labo = jnp.where(pos, labf, _F32(-1.0))
            lstage[pl.ds(out_base + base, 16)] = labo
            return 0

        return grp_body

    lax.fori_loop(0, npx // 16, make_grp_body(0), 0)

    if flush:
        pltpu.sync_copy(rstage.at[pl.ds(0, npx * 6)],
                        regr_hbm.at[b, pl.ds(p0 * 6, npx * 6)])
        pltpu.sync_copy(lstage.at[pl.ds(0, npx)],
                        lab_hbm.at[b, pl.ds(p0, npx)])


def _sc_body(gt_hbm, msw_hbm, regr_hbm, lab_hbm, boxes_v, msw_v,
             px1_v, py1_v, px2_v, py2_v, val_v,
             bx1_v, by1_v, bx2_v, by2_v, besta, besti, rstage, lstage):
    cid = lax.axis_index("c")
    sid = lax.axis_index("s")
    wid = sid * 2 + cid
    scr = (boxes_v, msw_v, px1_v, py1_v, px2_v, py2_v, val_v,
           bx1_v, by1_v, bx2_v, by2_v, besta, besti, rstage, lstage)
    hbm = (gt_hbm, msw_hbm, regr_hbm, lab_hbm)

    @pl.when(wid < 16)
    def _():
        b = lax.div(wid, 2)
        half = wid - b * 2
        _process_level(0, b, half * 32, 32, 0, True, *hbm, *scr)

    @pl.when((wid >= 16) & (wid < 24))
    def _():
        _process_level(1, wid - 16, wid * 0, 32, 0, True, *hbm, *scr)

    @pl.when(wid >= 24)
    def _():
        b = wid - 24
        z = b * 0
        _process_level(2, b, z, 16, 0, False, *hbm, *scr)
        _process_level(3, b, z, 8, 256, False, *hbm, *scr)
        _process_level(4, b, z, 4, 320, False, *hbm, *scr)
        pltpu.sync_copy(rstage.at[pl.ds(0, 512 * 6)],
                        regr_hbm.at[b, pl.ds(5120 * 6, 512 * 6)])
        pltpu.sync_copy(lstage.at[pl.ds(0, 512)],
                        lab_hbm.at[b, pl.ds(5120, 512)])


@jax.jit
def _sc_assign(gt_boxes, msw):
    mesh = plsc.VectorSubcoreMesh(core_axis_name="c", subcore_axis_name="s")
    f = pl.kernel(
        _sc_body,
        out_type=(
            jax.ShapeDtypeStruct((_B, _NPIXP * 6), _F32),
            jax.ShapeDtypeStruct((_B, _NPIXP), _F32),
        ),
        mesh=mesh,
        compiler_params=pltpu.CompilerParams(needs_layout_passes=False),
        scratch_types=[
            pltpu.VMEM((_N * 5,), _F32),
            pltpu.VMEM((_N * 5,), _F32),
            pltpu.VMEM((112,), _I32),
            pltpu.VMEM((112,), _I32),
            pltpu.VMEM((112,), _I32),
            pltpu.VMEM((112,), _I32),
            pltpu.VMEM((112,), _I32),
            pltpu.VMEM((112,), _F32),
            pltpu.VMEM((112,), _F32),
            pltpu.VMEM((112,), _F32),
            pltpu.VMEM((112,), _F32),
            pltpu.VMEM((2064,), _F32),
            pltpu.VMEM((2064,), _I32),
            pltpu.VMEM((2048 * 6 + 16,), _F32),
            pltpu.VMEM((2064,), _F32),
        ],
    )
    return f(gt_boxes.reshape(_B, _N * 5), msw.reshape(_B, _N * 5))


def _tc_finish_body(lab_ref, regr_ref, cls_ref, regrout_ref):
    regrout_ref[0] = regr_ref[0, :_NPIX, :]
    lab = lab_ref[0, 0, :_NPIX].astype(_I32)
    soft = regr_ref[0, :_NPIX, 4]
    mask = regr_ref[0, :_NPIX, 5]
    iot = lax.broadcasted_iota(_I32, (_NPIX, _NUM_CLASSES + 2), 1)
    oh = (lab[:, None] == iot).astype(_F32)
    cls = jnp.where(iot == _NUM_CLASSES, soft[:, None],
                    jnp.where(iot == _NUM_CLASSES + 1, mask[:, None], oh))
    cls_ref[0] = cls


@jax.jit
def _tc_finish(lab, regr_flat):
    return pl.pallas_call(
        _tc_finish_body,
        grid=(_B,),
        in_specs=[
            pl.BlockSpec((1, 1, _NPIXP), lambda i: (i, 0, 0)),
            pl.BlockSpec((1, _NPIXP, 6), lambda i: (i, 0, 0)),
        ],
        out_specs=[
            pl.BlockSpec((1, _NPIX, _NUM_CLASSES + 2), lambda i: (i, 0, 0)),
            pl.BlockSpec((1, _NPIX, 6), lambda i: (i, 0, 0)),
        ],
        out_shape=[
            jax.ShapeDtypeStruct((_B, _NPIX, _NUM_CLASSES + 2), _F32),
            jax.ShapeDtypeStruct((_B, _NPIX, 6), _F32),
        ],
    )(lab.reshape(_B, 1, _NPIXP), regr_flat.reshape(_B, _NPIXP, 6))


def kernel(fm_shapes, gt_boxes, meta_select_weight):
    del fm_shapes
    regr_flat, lab = _sc_assign(gt_boxes, meta_select_weight)
    cls_t, regr_t = _tc_finish(lab, regr_flat)
    return cls_t, regr_t


# speedup vs baseline: 1.9802x; 1.7656x over previous
import functools

import jax
import jax.numpy as jnp
from jax import lax
from jax.experimental import pallas as pl
from jax.experimental.pallas import tpu as pltpu
from jax.experimental.pallas import tpu_sc as plsc

_NUM_CLASSES = 80
_SHRINK = 0.2
_B = 8
_N = 100
_LEVELS = (
    (8, 64, 64, 6, 0),
    (16, 32, 32, 5, 4096),
    (32, 16, 16, 4, 5120),
    (64, 8, 8, 3, 5376),
    (128, 4, 4, 2, 5440),
)
_NPIX = 5456
_NPIXP = 5632
_F32 = jnp.float32
_I32 = jnp.int32


def _iota16():
    return lax.iota(_I32, 16)


def _csplat(c):
    return jnp.full((16,), c, dtype=_I32)


_SOFF = 2048


def _process_level(level, b, y0, nrows, out_base, flush,
                   gt_hbm, msw_hbm, regr_hbm, lab_hbm,
                   boxes_v, msw_v, px1_v, py1_v, px2_v, py2_v, val_v,
                   bx1_v, by1_v, bx2_v, by2_v, besta, besti, rstage, lstage):
    stride, fh, fw, log2fw, p0_level = _LEVELS[level]
    npx = nrows * fw
    stride_f = _F32(stride)
    iot = _iota16()

    pltpu.sync_copy(gt_hbm.at[b], boxes_v)
    pltpu.sync_copy(msw_hbm.at[b], msw_v)

    for g in range(7):
        ridx = g * 16 + iot
        gmask = ridx < _N
        fidx = ridx * 5
        x1 = plsc.load_gather(boxes_v, [fidx], mask=gmask)
        y1 = plsc.load_gather(boxes_v, [fidx + 1], mask=gmask)
        x2 = plsc.load_gather(boxes_v, [fidx + 2], mask=gmask)
        y2 = plsc.load_gather(boxes_v, [fidx + 3], mask=gmask)
        valid = ((jnp.abs(x1) + jnp.abs(y1) + jnp.abs(x2) + jnp.abs(y2)) > 0.0) & gmask
        cx = (x1 + x2) * _F32(0.5)
        cy = (y1 + y2) * _F32(0.5)
        w = x2 - x1
        h = y2 - y1
        inv_s = _F32(1.0 / stride)
        sx1 = (cx - w * _F32(_SHRINK) * _F32(0.5)) * inv_s
        sy1 = (cy - h * _F32(_SHRINK) * _F32(0.5)) * inv_s
        sx2 = (cx + w * _F32(_SHRINK) * _F32(0.5)) * inv_s
        sy2 = (cy + h * _F32(_SHRINK) * _F32(0.5)) * inv_s
        p1 = jnp.clip(sx1.astype(_I32), 0, fw - 1)
        q1 = jnp.clip(sy1.astype(_I32), 0, fh - 1)
        c2 = sx2.astype(_I32)
        c2 = c2 + jnp.where(c2.astype(_F32) < sx2, 1, 0)
        r2 = sy2.astype(_I32)
        r2 = r2 + jnp.where(r2.astype(_F32) < sy2, 1, 0)
        p2 = jnp.clip(c2, p1 + 1, fw)
        q2 = jnp.clip(r2, q1 + 1, fh)
        sl = pl.ds(g * 16, 16)
        px1_v[sl] = p1
        py1_v[sl] = q1
        px2_v[sl] = p2
        py2_v[sl] = q2
        val_v[sl] = jnp.where(valid, 1, 0)
        bx1_v[sl] = x1
        by1_v[sl] = y1
        bx2_v[sl] = x2
        by2_v[sl] = y2

    big = jnp.full((16,), 1e7, dtype=_F32)
    zer = jnp.zeros((16,), dtype=_I32)

    def init_body(g, _):
        besta[pl.ds(g * 16, 16)] = big
        besti[pl.ds(g * 16, 16)] = zer
        return 0

    lax.fori_loop(0, npx // 16, init_body, 0)

    def box_grp_body(g, _):
        gs = pl.ds(g * 16, 16)
        p1v = px1_v[gs]
        p2v = px2_v[gs]
        q1v = py1_v[gs]
        q2v = py2_v[gs]
        vv = val_v[gs]
        x1v = bx1_v[gs]
        y1v = by1_v[gs]
        x2v = bx2_v[gs]
        y2v = by2_v[gs]
        for j in range(16):
            n = g * 16 + j
            v = vv[j]
            p1 = p1v[j]
            q1 = q1v[j]
            q2 = q2v[j]
            x1f = x1v[j]
            y1f = y1v[j]
            x2f = x2v[j]
            y2f = y2v[j]
            ry1 = jnp.maximum(q1, y0)
            ry2 = jnp.minimum(q2, y0 + nrows)
            ry2 = jnp.where(v > 0, jnp.maximum(ry2, ry1), ry1)
            rw = p2v[j] - p1
            m_in = iot < rw
            xsf = (p1 + iot).astype(_F32)
            sx = (xsf + _F32(0.5)) * stride_f
            dl = jnp.maximum(sx - x1f, 0.0)
            dr = jnp.maximum(x2f - sx, 0.0)
            dlr = dl + dr
            nvec = lax.broadcast(n, (16,))

            def row_body(y, _, dlr=dlr, m_in=m_in, y1f=y1f, y2f=y2f,
                         p1=p1, nvec=nvec):
                sy = (y.astype(_F32) + _F32(0.5)) * stride_f
                dt = jnp.maximum(sy - y1f, 0.0)
                db = jnp.maximum(y2f - sy, 0.0)
                area = dlr * (dt + db)
                loc = (y - y0) * fw + p1
                sl = pl.ds(loc, 16)
                cur = besta[sl]
                upd = m_in & (area < cur)
                besta[sl] = jnp.where(upd, area, cur)
                curi = besti[sl]
                besti[sl] = jnp.where(upd, nvec, curi)
                return 0

            lax.fori_loop(ry1, ry2, row_body, 0)
        return 0

    lax.fori_loop(0, 7, box_grp_body, 0)

    inv4s = _F32(1.0 / (4.0 * stride))
    p0 = p0_level + y0 * fw

    def make_grp_body(cls_off):
        def grp_body(g, _):
            base = g * 16
            lp = base + iot
            sl = pl.ds(base, 16)
            idxv = besti[sl]
            areav = besta[sl]
            pos = areav < 1e7
            posf = jnp.where(pos, _F32(1.0), _F32(0.0))
            x = lp & (fw - 1)
            y = y0 + lax.shift_right_logical(lp, log2fw)
            sx = (x.astype(_F32) + _F32(0.5)) * stride_f
            sy = (y.astype(_F32) + _F32(0.5)) * stride_f
            idx5 = idxv * 5
            bx1 = plsc.load_gather(boxes_v, [idx5])
            by1 = plsc.load_gather(boxes_v, [idx5 + 1])
            bx2 = plsc.load_gather(boxes_v, [idx5 + 2])
            by2 = plsc.load_gather(boxes_v, [idx5 + 3])
            labf = plsc.load_gather(boxes_v, [idx5 + 4])
            mw = plsc.load_gather(msw_v, [idx5 + level])
            dl = jnp.maximum(sx - bx1, 0.0)
            dt = jnp.maximum(sy - by1, 0.0)
            dr = jnp.maximum(bx2 - sx, 0.0)
            db = jnp.maximum(by2 - sy, 0.0)
            apn = jnp.minimum(dl, dr) * jnp.minimum(dt, db)
            apd = jnp.maximum(jnp.maximum(dl, dr) * jnp.maximum(dt, db),
                              1e-12)
            soft = jnp.where(pos, (apn / apd) * mw, _F32(1.0))
            ob = out_base + base
            rstage[pl.ds(0 * _SOFF + ob, 16)] = dl * inv4s * posf
            rstage[pl.ds(1 * _SOFF + ob, 16)] = dt * inv4s * posf
            rstage[pl.ds(2 * _SOFF + ob, 16)] = dr * inv4s * posf
            rstage[pl.ds(3 * _SOFF + ob, 16)] = db * inv4s * posf
            rstage[pl.ds(4 * _SOFF + ob, 16)] = soft
            rstage[pl.ds(5 * _SOFF + ob, 16)] = posf
            labo = jnp.where(pos, labf.astype(_I32), -1)
            lstage[pl.ds(ob, 16)] = labo
            return 0

        return grp_body

    lax.fori_loop(0, npx // 16, make_grp_body(0), 0)

    if flush:
        for c in range(6):
            pltpu.sync_copy(rstage.at[pl.ds(c * _SOFF, npx)],
                            regr_hbm.at[c, b, pl.ds(p0, npx)])
        pltpu.sync_copy(lstage.at[pl.ds(0, npx)],
                        lab_hbm.at[0, b, pl.ds(p0, npx)])


def _sc_body(gt_hbm, msw_hbm, regr_hbm, lab_hbm, boxes_v, msw_v,
             px1_v, py1_v, px2_v, py2_v, val_v,
             bx1_v, by1_v, bx2_v, by2_v, besta, besti, rstage, lstage):
    cid = lax.axis_index("c")
    sid = lax.axis_index("s")
    wid = sid * 2 + cid
    scr = (boxes_v, msw_v, px1_v, py1_v, px2_v, py2_v, val_v,
           bx1_v, by1_v, bx2_v, by2_v, besta, besti, rstage, lstage)
    hbm = (gt_hbm, msw_hbm, regr_hbm, lab_hbm)

    @pl.when(wid < 16)
    def _():
        b = lax.div(wid, 2)
        half = wid - b * 2
        _process_level(0, b, half * 32, 32, 0, True, *hbm, *scr)

    @pl.when((wid >= 16) & (wid < 24))
    def _():
        _process_level(1, wid - 16, wid * 0, 32, 0, True, *hbm, *scr)

    @pl.when(wid >= 24)
    def _():
        b = wid - 24
        z = b * 0
        _process_level(2, b, z, 16, 0, False, *hbm, *scr)
        _process_level(3, b, z, 8, 256, False, *hbm, *scr)
        _process_level(4, b, z, 4, 320, False, *hbm, *scr)
        for c in range(6):
            pltpu.sync_copy(rstage.at[pl.ds(c * _SOFF, 512)],
                            regr_hbm.at[c, b, pl.ds(5120, 512)])
        pltpu.sync_copy(lstage.at[pl.ds(0, 512)],
                        lab_hbm.at[0, b, pl.ds(5120, 512)])


@jax.jit
def _sc_assign(gt_boxes, msw):
    mesh = plsc.VectorSubcoreMesh(core_axis_name="c", subcore_axis_name="s")
    f = pl.kernel(
        _sc_body,
        out_type=(
            jax.ShapeDtypeStruct((6, _B, _NPIXP), _F32),
            jax.ShapeDtypeStruct((1, _B, _NPIXP), _I32),
        ),
        mesh=mesh,
        compiler_params=pltpu.CompilerParams(needs_layout_passes=False),
        scratch_types=[
            pltpu.VMEM((_N * 5,), _F32),
            pltpu.VMEM((_N * 5,), _F32),
            pltpu.VMEM((112,), _I32),
            pltpu.VMEM((112,), _I32),
            pltpu.VMEM((112,), _I32),
            pltpu.VMEM((112,), _I32),
            pltpu.VMEM((112,), _I32),
            pltpu.VMEM((112,), _F32),
            pltpu.VMEM((112,), _F32),
            pltpu.VMEM((112,), _F32),
            pltpu.VMEM((112,), _F32),
            pltpu.VMEM((2064,), _F32),
            pltpu.VMEM((2064,), _I32),
            pltpu.VMEM((_SOFF * 6 + 16,), _F32),
            pltpu.VMEM((2064,), _I32),
        ],
    )
    return f(gt_boxes.reshape(_B, _N * 5), msw.reshape(_B, _N * 5))


def _tc_finish_body(lab_ref, sm_ref, cls_ref, regr_ref):
    c = pl.program_id(0)

    @pl.when(c < _NUM_CLASSES + 2)
    def _():
        labi = lab_ref[0, :, :_NPIX]
        oh = (labi == c).astype(_F32)
        sm = sm_ref[0, :, :_NPIX]
        cls_ref[0] = jnp.where(c < _NUM_CLASSES, oh, sm)

    @pl.when(c >= _NUM_CLASSES + 2)
    def _():
        regr_ref[0] = sm_ref[0, :, :_NPIX]


@jax.jit
def _tc_finish(lab, regr_soa):
    nc2 = _NUM_CLASSES + 2
    return pl.pallas_call(
        _tc_finish_body,
        grid=(nc2 + 6,),
        in_specs=[
            pl.BlockSpec((1, _B, _NPIXP), lambda c: (0, 0, 0)),
            pl.BlockSpec(
                (1, _B, _NPIXP),
                lambda c: (jnp.where(c < nc2 - 1, 4,
                                     jnp.where(c == nc2 - 1, 5, c - nc2)),
                           0, 0)),
        ],
        out_specs=[
            pl.BlockSpec((1, _B, _NPIX),
                         lambda c: (jnp.minimum(c, nc2 - 1), 0, 0)),
            pl.BlockSpec((1, _B, _NPIX),
                         lambda c: (jnp.maximum(c - nc2, 0), 0, 0)),
        ],
        out_shape=[
            jax.ShapeDtypeStruct((nc2, _B, _NPIX), _F32),
            jax.ShapeDtypeStruct((6, _B, _NPIX), _F32),
        ],
    )(lab, regr_soa)


def kernel(fm_shapes, gt_boxes, meta_select_weight):
    del fm_shapes
    regr_soa, lab = _sc_assign(gt_boxes, meta_select_weight)
    cls_p, regr_p = _tc_finish(lab, regr_soa)
    cls_t = jnp.transpose(cls_p, (1, 2, 0))
    regr_t = jnp.transpose(regr_p, (1, 2, 0))
    return cls_t, regr_t


# speedup vs baseline: 2.8623x; 1.4455x over previous
import functools

import jax
import jax.numpy as jnp
from jax import lax
from jax.experimental import pallas as pl
from jax.experimental.pallas import tpu as pltpu
from jax.experimental.pallas import tpu_sc as plsc

_NUM_CLASSES = 80
_SHRINK = 0.2
_B = 8
_N = 100
_LEVELS = (
    (8, 64, 64, 6, 0),
    (16, 32, 32, 5, 4096),
    (32, 16, 16, 4, 5120),
    (64, 8, 8, 3, 5376),
    (128, 4, 4, 2, 5440),
)
_NPIX = 5456
_NPIXP = 5632
_F32 = jnp.float32
_I32 = jnp.int32


def _iota16():
    return lax.iota(_I32, 16)


def _csplat(c):
    return jnp.full((16,), c, dtype=_I32)


_SOFF = 2048


def _process_level(level, b, y0, nrows, out_base, flush,
                   gt_hbm, msw_hbm, regr_hbm, lab_hbm,
                   boxes_v, msw_v, px1_v, py1_v, px2_v, py2_v, val_v,
                   bx1_v, by1_v, bx2_v, by2_v, besta, besti, rstage, lstage):
    stride, fh, fw, log2fw, p0_level = _LEVELS[level]
    npx = nrows * fw
    stride_f = _F32(stride)
    iot = _iota16()

    pltpu.sync_copy(gt_hbm.at[b], boxes_v)
    pltpu.sync_copy(msw_hbm.at[b], msw_v)

    for g in range(7):
        ridx = g * 16 + iot
        gmask = ridx < _N
        fidx = ridx * 5
        x1 = plsc.load_gather(boxes_v, [fidx], mask=gmask)
        y1 = plsc.load_gather(boxes_v, [fidx + 1], mask=gmask)
        x2 = plsc.load_gather(boxes_v, [fidx + 2], mask=gmask)
        y2 = plsc.load_gather(boxes_v, [fidx + 3], mask=gmask)
        valid = ((jnp.abs(x1) + jnp.abs(y1) + jnp.abs(x2) + jnp.abs(y2)) > 0.0) & gmask
        cx = (x1 + x2) * _F32(0.5)
        cy = (y1 + y2) * _F32(0.5)
        w = x2 - x1
        h = y2 - y1
        inv_s = _F32(1.0 / stride)
        sx1 = (cx - w * _F32(_SHRINK) * _F32(0.5)) * inv_s
        sy1 = (cy - h * _F32(_SHRINK) * _F32(0.5)) * inv_s
        sx2 = (cx + w * _F32(_SHRINK) * _F32(0.5)) * inv_s
        sy2 = (cy + h * _F32(_SHRINK) * _F32(0.5)) * inv_s
        p1 = jnp.clip(sx1.astype(_I32), 0, fw - 1)
        q1 = jnp.clip(sy1.astype(_I32), 0, fh - 1)
        c2 = sx2.astype(_I32)
        c2 = c2 + jnp.where(c2.astype(_F32) < sx2, 1, 0)
        r2 = sy2.astype(_I32)
        r2 = r2 + jnp.where(r2.astype(_F32) < sy2, 1, 0)
        p2 = jnp.clip(c2, p1 + 1, fw)
        q2 = jnp.clip(r2, q1 + 1, fh)
        sl = pl.ds(g * 16, 16)
        px1_v[sl] = p1
        py1_v[sl] = q1
        px2_v[sl] = p2
        py2_v[sl] = q2
        val_v[sl] = jnp.where(valid, 1, 0)
        bx1_v[sl] = x1
        by1_v[sl] = y1
        bx2_v[sl] = x2
        by2_v[sl] = y2

    big = jnp.full((16,), 1e7, dtype=_F32)
    zer = jnp.zeros((16,), dtype=_I32)

    def init_body(g, _):
        besta[pl.ds(g * 16, 16)] = big
        besti[pl.ds(g * 16, 16)] = zer
        return 0

    lax.fori_loop(0, npx // 16, init_body, 0)

    def box_grp_body(g, _):
        gs = pl.ds(g * 16, 16)
        p1v = px1_v[gs]
        p2v = px2_v[gs]
        q1v = py1_v[gs]
        q2v = py2_v[gs]
        vv = val_v[gs]
        x1v = bx1_v[gs]
        y1v = by1_v[gs]
        x2v = bx2_v[gs]
        y2v = by2_v[gs]
        for j in range(16):
            n = g * 16 + j
            v = vv[j]
            p1 = p1v[j]
            q1 = q1v[j]
            q2 = q2v[j]
            x1f = x1v[j]
            y1f = y1v[j]
            x2f = x2v[j]
            y2f = y2v[j]
            ry1 = jnp.maximum(q1, y0)
            ry2 = jnp.minimum(q2, y0 + nrows)
            ry2 = jnp.where(v > 0, jnp.maximum(ry2, ry1), ry1)
            rw = p2v[j] - p1
            m_in = iot < rw
            xsf = (p1 + iot).astype(_F32)
            sx = (xsf + _F32(0.5)) * stride_f
            dl = jnp.maximum(sx - x1f, 0.0)
            dr = jnp.maximum(x2f - sx, 0.0)
            dlr = dl + dr
            nvec = lax.broadcast(n, (16,))

            def row_body(y, _, dlr=dlr, m_in=m_in, y1f=y1f, y2f=y2f,
                         p1=p1, nvec=nvec):
                sy = (y.astype(_F32) + _F32(0.5)) * stride_f
                dt = jnp.maximum(sy - y1f, 0.0)
                db = jnp.maximum(y2f - sy, 0.0)
                area = dlr * (dt + db)
                loc = (y - y0) * fw + p1
                sl = pl.ds(loc, 16)
                cur = besta[sl]
                upd = m_in & (area < cur)
                besta[sl] = jnp.where(upd, area, cur)
                curi = besti[sl]
                besti[sl] = jnp.where(upd, nvec, curi)
                return 0

            lax.fori_loop(ry1, ry2, row_body, 0)
        return 0

    lax.fori_loop(0, 7, box_grp_body, 0)

    inv4s = _F32(1.0 / (4.0 * stride))
    p0 = p0_level + y0 * fw

    def make_grp_body(cls_off):
        def grp_body(g, _):
            base = g * 16
            lp = base + iot
            sl = pl.ds(base, 16)
            idxv = besti[sl]
            areav = besta[sl]
            pos = areav < 1e7
            posf = jnp.where(pos, _F32(1.0), _F32(0.0))
            x = lp & (fw - 1)
            y = y0 + lax.shift_right_logical(lp, log2fw)
            sx = (x.astype(_F32) + _F32(0.5)) * stride_f
            sy = (y.astype(_F32) + _F32(0.5)) * stride_f
            idx5 = idxv * 5
            bx1 = plsc.load_gather(boxes_v, [idx5])
            by1 = plsc.load_gather(boxes_v, [idx5 + 1])
            bx2 = plsc.load_gather(boxes_v, [idx5 + 2])
            by2 = plsc.load_gather(boxes_v, [idx5 + 3])
            labf = plsc.load_gather(boxes_v, [idx5 + 4])
            mw = plsc.load_gather(msw_v, [idx5 + level])
            dl = jnp.maximum(sx - bx1, 0.0)
            dt = jnp.maximum(sy - by1, 0.0)
            dr = jnp.maximum(bx2 - sx, 0.0)
            db = jnp.maximum(by2 - sy, 0.0)
            apn = jnp.minimum(dl, dr) * jnp.minimum(dt, db)
            apd = jnp.maximum(jnp.maximum(dl, dr) * jnp.maximum(dt, db),
                              1e-12)
            soft = jnp.where(pos, (apn / apd) * mw, _F32(1.0))
            ob = out_base + base
            rstage[pl.ds(0 * _SOFF + ob, 16)] = dl * inv4s * posf
            rstage[pl.ds(1 * _SOFF + ob, 16)] = dt * inv4s * posf
            rstage[pl.ds(2 * _SOFF + ob, 16)] = dr * inv4s * posf
            rstage[pl.ds(3 * _SOFF + ob, 16)] = db * inv4s * posf
            rstage[pl.ds(4 * _SOFF + ob, 16)] = soft
            rstage[pl.ds(5 * _SOFF + ob, 16)] = posf
            labo = jnp.where(pos, labf.astype(_I32), -1)
            lstage[pl.ds(ob, 16)] = labo
            return 0

        return grp_body

    lax.fori_loop(0, npx // 16, make_grp_body(0), 0)

    if flush:
        for c in range(6):
            pltpu.sync_copy(rstage.at[pl.ds(c * _SOFF, npx)],
                            regr_hbm.at[c, b, pl.ds(p0, npx)])
        pltpu.sync_copy(lstage.at[pl.ds(0, npx)],
                        lab_hbm.at[0, b, pl.ds(p0, npx)])


def _sc_body(gt_hbm, msw_hbm, regr_hbm, lab_hbm, boxes_v, msw_v,
             px1_v, py1_v, px2_v, py2_v, val_v,
             bx1_v, by1_v, bx2_v, by2_v, besta, besti, rstage, lstage):
    cid = lax.axis_index("c")
    sid = lax.axis_index("s")
    wid = sid * 2 + cid
    scr = (boxes_v, msw_v, px1_v, py1_v, px2_v, py2_v, val_v,
           bx1_v, by1_v, bx2_v, by2_v, besta, besti, rstage, lstage)
    hbm = (gt_hbm, msw_hbm, regr_hbm, lab_hbm)

    @pl.when(wid < 16)
    def _():
        b = lax.div(wid, 2)
        half = wid - b * 2
        _process_level(0, b, half * 32, 32, 0, True, *hbm, *scr)

    @pl.when((wid >= 16) & (wid < 24))
    def _():
        _process_level(1, wid - 16, wid * 0, 32, 0, True, *hbm, *scr)

    @pl.when(wid >= 24)
    def _():
        b = wid - 24
        z = b * 0
        _process_level(2, b, z, 16, 0, False, *hbm, *scr)
        _process_level(3, b, z, 8, 256, False, *hbm, *scr)
        _process_level(4, b, z, 4, 320, False, *hbm, *scr)
        for c in range(6):
            pltpu.sync_copy(rstage.at[pl.ds(c * _SOFF, 512)],
                            regr_hbm.at[c, b, pl.ds(5120, 512)])
        pltpu.sync_copy(lstage.at[pl.ds(0, 512)],
                        lab_hbm.at[0, b, pl.ds(5120, 512)])


@jax.jit
def _sc_assign(gt_boxes, msw):
    mesh = plsc.VectorSubcoreMesh(core_axis_name="c", subcore_axis_name="s")
    f = pl.kernel(
        _sc_body,
        out_type=(
            jax.ShapeDtypeStruct((6, _B, _NPIXP), _F32),
            jax.ShapeDtypeStruct((1, _B, _NPIXP), _I32),
        ),
        mesh=mesh,
        compiler_params=pltpu.CompilerParams(needs_layout_passes=False),
        scratch_types=[
            pltpu.VMEM((_N * 5,), _F32),
            pltpu.VMEM((_N * 5,), _F32),
            pltpu.VMEM((112,), _I32),
            pltpu.VMEM((112,), _I32),
            pltpu.VMEM((112,), _I32),
            pltpu.VMEM((112,), _I32),
            pltpu.VMEM((112,), _I32),
            pltpu.VMEM((112,), _F32),
            pltpu.VMEM((112,), _F32),
            pltpu.VMEM((112,), _F32),
            pltpu.VMEM((112,), _F32),
            pltpu.VMEM((2064,), _F32),
            pltpu.VMEM((2064,), _I32),
            pltpu.VMEM((_SOFF * 6 + 16,), _F32),
            pltpu.VMEM((2064,), _I32),
        ],
    )
    return f(gt_boxes.reshape(_B, _N * 5), msw.reshape(_B, _N * 5))


def _tc_finish_body(lab_ref, sm_ref, cls_ref, regr_ref):
    labi = lab_ref[0, :, :_NPIX]
    for c in range(_NUM_CLASSES):
        cls_ref[c] = (labi == c).astype(_F32)
    cls_ref[_NUM_CLASSES] = sm_ref[4, :, :_NPIX]
    cls_ref[_NUM_CLASSES + 1] = sm_ref[5, :, :_NPIX]
    regr_ref[...] = sm_ref[:, :, :_NPIX]


@jax.jit
def _tc_finish(lab, regr_soa):
    nc2 = _NUM_CLASSES + 2
    return pl.pallas_call(
        _tc_finish_body,
        out_shape=[
            jax.ShapeDtypeStruct((nc2, _B, _NPIX), _F32),
            jax.ShapeDtypeStruct((6, _B, _NPIX), _F32),
        ],
        compiler_params=pltpu.CompilerParams(
            vmem_limit_bytes=100 * 1024 * 1024),
    )(lab, regr_soa)


def kernel(fm_shapes, gt_boxes, meta_select_weight):
    del fm_shapes
    regr_soa, lab = _sc_assign(gt_boxes, meta_select_weight)
    cls_p, regr_p = _tc_finish(lab, regr_soa)
    cls_t = jnp.transpose(cls_p, (1, 2, 0))
    regr_t = jnp.transpose(regr_p, (1, 2, 0))
    return cls_t, regr_t


# speedup vs baseline: 3.6700x; 1.2822x over previous
import jax
import jax.numpy as jnp
from jax import lax
from jax.experimental import pallas as pl
from jax.experimental.pallas import tpu as pltpu
from jax.experimental.pallas import tpu_sc as plsc

_NUM_CLASSES = 80
_SHRINK = 0.2
_B = 8
_N = 100
_NPIX = 5456
_NPIXP = 5632
_F32 = jnp.float32
_I32 = jnp.int32
_SOFF = 2048


def _unit_body(lvl, b, y0, nrows, out_base,
               gt_hbm, msw_hbm,
               boxes_v, msw_v, px1_v, py1_v, px2_v, py2_v, val_v,
               bx1_v, by1_v, bx2_v, by2_v, besta, besti, rstage, lstage):
    iot = lax.iota(_I32, 16)
    log2fw = 6 - lvl
    fw = lax.shift_left(1, log2fw)
    fh = fw
    stride = lax.shift_left(8, lvl)
    stride_f = stride.astype(_F32)
    inv_s = jnp.full((16,), 1.0, _F32) / lax.broadcast(stride_f, (16,))
    inv4s = inv_s * _F32(0.25)
    npx = nrows * fw

    def setup_body(g, _):
        ridx = g * 16 + iot
        gmask = ridx < _N
        fidx = ridx * 5
        x1 = plsc.load_gather(boxes_v, [fidx], mask=gmask)
        y1 = plsc.load_gather(boxes_v, [fidx + 1], mask=gmask)
        x2 = plsc.load_gather(boxes_v, [fidx + 2], mask=gmask)
        y2 = plsc.load_gather(boxes_v, [fidx + 3], mask=gmask)
        valid = ((jnp.abs(x1) + jnp.abs(y1) + jnp.abs(x2) + jnp.abs(y2))
                 > 0.0) & gmask
        cx = (x1 + x2) * _F32(0.5)
        cy = (y1 + y2) * _F32(0.5)
        w = x2 - x1
        h = y2 - y1
        sx1 = (cx - w * _F32(_SHRINK) * _F32(0.5)) * inv_s
        sy1 = (cy - h * _F32(_SHRINK) * _F32(0.5)) * inv_s
        sx2 = (cx + w * _F32(_SHRINK) * _F32(0.5)) * inv_s
        sy2 = (cy + h * _F32(_SHRINK) * _F32(0.5)) * inv_s
        p1 = jnp.clip(sx1.astype(_I32), 0, fw - 1)
        q1 = jnp.clip(sy1.astype(_I32), 0, fh - 1)
        c2 = sx2.astype(_I32)
        c2 = c2 + jnp.where(c2.astype(_F32) < sx2, 1, 0)
        r2 = sy2.astype(_I32)
        r2 = r2 + jnp.where(r2.astype(_F32) < sy2, 1, 0)
        p2 = jnp.clip(c2, p1 + 1, fw)
        q2 = jnp.clip(r2, q1 + 1, fh)
        sl = pl.ds(g * 16, 16)
        px1_v[sl] = p1
        py1_v[sl] = q1
        px2_v[sl] = p2
        py2_v[sl] = q2
        val_v[sl] = jnp.where(valid, 1, 0)
        bx1_v[sl] = x1
        by1_v[sl] = y1
        bx2_v[sl] = x2
        by2_v[sl] = y2
        return 0

    lax.fori_loop(0, 7, setup_body, 0)

    big = jnp.full((16,), 1e7, dtype=_F32)
    zer = jnp.zeros((16,), dtype=_I32)

    def init_body(g, _):
        besta[pl.ds(g * 16, 16)] = big
        besti[pl.ds(g * 16, 16)] = zer
        return 0

    lax.fori_loop(0, lax.shift_right_logical(npx, 4), init_body, 0)

    def box_grp_body(g, _):
        gs = pl.ds(g * 16, 16)
        p1v = px1_v[gs]
        p2v = px2_v[gs]
        q1v = py1_v[gs]
        q2v = py2_v[gs]
        vv = val_v[gs]
        x1v = bx1_v[gs]
        y1v = by1_v[gs]
        x2v = bx2_v[gs]
        y2v = by2_v[gs]
        for j in range(16):
            n = g * 16 + j
            v = vv[j]
            p1 = p1v[j]
            q1 = q1v[j]
            q2 = q2v[j]
            x1f = x1v[j]
            y1f = y1v[j]
            x2f = x2v[j]
            y2f = y2v[j]
            ry1 = jnp.maximum(q1, y0)
            ry2 = jnp.minimum(q2, y0 + nrows)
            ry2 = jnp.where(v > 0, jnp.maximum(ry2, ry1), ry1)
            rw = p2v[j] - p1
            m_in = iot < rw
            xsf = (p1 + iot).astype(_F32)
            sx = (xsf + _F32(0.5)) * stride_f
            dl = jnp.maximum(sx - x1f, 0.0)
            dr = jnp.maximum(x2f - sx, 0.0)
            dlr = dl + dr
            nvec = lax.broadcast(n, (16,))

            def row_body(y, _, dlr=dlr, m_in=m_in, y1f=y1f, y2f=y2f,
                         p1=p1, nvec=nvec):
                sy = (y.astype(_F32) + _F32(0.5)) * stride_f
                dt = jnp.maximum(sy - y1f, 0.0)
                db = jnp.maximum(y2f - sy, 0.0)
                area = dlr * (dt + db)
                loc = (y - y0) * fw + p1
                sl = pl.ds(loc, 16)
                cur = besta[sl]
                upd = m_in & (area < cur)
                besta[sl] = jnp.where(upd, area, cur)
                curi = besti[sl]
                besti[sl] = jnp.where(upd, nvec, curi)
                return 0

            lax.fori_loop(ry1, ry2, row_body, 0)
        return 0

    lax.fori_loop(0, 7, box_grp_body, 0)

    def grp_body(g, _):
        base = g * 16
        lp = base + iot
        sl = pl.ds(base, 16)
        idxv = besti[sl]
        areav = besta[sl]
        pos = areav < 1e7
        posf = jnp.where(pos, _F32(1.0), _F32(0.0))
        x = lp & (fw - 1)
        y = y0 + lax.shift_right_logical(lp, log2fw)
        sx = (x.astype(_F32) + _F32(0.5)) * stride_f
        sy = (y.astype(_F32) + _F32(0.5)) * stride_f
        idx5 = idxv * 5
        bx1 = plsc.load_gather(boxes_v, [idx5])
        by1 = plsc.load_gather(boxes_v, [idx5 + 1])
        bx2 = plsc.load_gather(boxes_v, [idx5 + 2])
        by2 = plsc.load_gather(boxes_v, [idx5 + 3])
        labf = plsc.load_gather(boxes_v, [idx5 + 4])
        mw = plsc.load_gather(msw_v, [idx5 + lvl])
        dl = jnp.maximum(sx - bx1, 0.0)
        dt = jnp.maximum(sy - by1, 0.0)
        dr = jnp.maximum(bx2 - sx, 0.0)
        db = jnp.maximum(by2 - sy, 0.0)
        apn = jnp.minimum(dl, dr) * jnp.minimum(dt, db)
        apd = jnp.maximum(jnp.maximum(dl, dr) * jnp.maximum(dt, db), 1e-12)
        soft = jnp.where(pos, (apn / apd) * mw, _F32(1.0))
        ob = out_base + base
        rstage[pl.ds(0 * _SOFF + ob, 16)] = dl * inv4s * posf
        rstage[pl.ds(1 * _SOFF + ob, 16)] = dt * inv4s * posf
        rstage[pl.ds(2 * _SOFF + ob, 16)] = dr * inv4s * posf
        rstage[pl.ds(3 * _SOFF + ob, 16)] = db * inv4s * posf
        rstage[pl.ds(4 * _SOFF + ob, 16)] = soft
        rstage[pl.ds(5 * _SOFF + ob, 16)] = posf
        labo = jnp.where(pos, labf.astype(_I32), -1)
        lstage[pl.ds(ob, 16)] = labo
        return 0

    lax.fori_loop(0, lax.shift_right_logical(npx, 4), grp_body, 0)


def _sc_body(gt_hbm, msw_hbm, regr_hbm, lab_hbm, boxes_v, msw_v,
             px1_v, py1_v, px2_v, py2_v, val_v,
             bx1_v, by1_v, bx2_v, by2_v, besta, besti, rstage, lstage):
    cid = lax.axis_index("c")
    sid = lax.axis_index("s")
    wid = sid * 2 + cid
    scr = (boxes_v, msw_v, px1_v, py1_v, px2_v, py2_v, val_v,
           bx1_v, by1_v, bx2_v, by2_v, besta, besti, rstage, lstage)

    is_a = wid < 16
    is_b = (wid >= 16) & (wid < 24)
    b = jnp.where(is_a, lax.div(wid, 2),
                  jnp.where(is_b, wid - 16, wid - 24))
    nunits = jnp.where(wid < 24, 1, 3)

    pltpu.sync_copy(gt_hbm.at[b], boxes_v)
    pltpu.sync_copy(msw_hbm.at[b], msw_v)

    def unit(u, _):
        lvl = jnp.where(is_a, 0, jnp.where(is_b, 1, 2 + u))
        y0 = jnp.where(is_a, (wid & 1) * 32, 0)
        fh = lax.shift_left(1, 6 - lvl)
        nrows = jnp.where(is_a, 32, fh)
        out_base = jnp.where(u == 1, 256, jnp.where(u == 2, 320, 0))
        _unit_body(lvl, b, y0, nrows, out_base, gt_hbm, msw_hbm, *scr)
        return 0

    lax.fori_loop(0, nunits, unit, 0)

    @pl.when(is_a)
    def _():
        p0 = (wid & 1) * 2048
        for c in range(6):
            pltpu.sync_copy(rstage.at[pl.ds(c * _SOFF, 2048)],
                            regr_hbm.at[c, b, pl.ds(p0, 2048)])
        pltpu.sync_copy(lstage.at[pl.ds(0, 2048)],
                        lab_hbm.at[0, b, pl.ds(p0, 2048)])

    @pl.when(is_b)
    def _():
        for c in range(6):
            pltpu.sync_copy(rstage.at[pl.ds(c * _SOFF, 1024)],
                            regr_hbm.at[c, b, pl.ds(4096, 1024)])
        pltpu.sync_copy(lstage.at[pl.ds(0, 1024)],
                        lab_hbm.at[0, b, pl.ds(4096, 1024)])

    @pl.when(wid >= 24)
    def _():
        for c in range(6):
            pltpu.sync_copy(rstage.at[pl.ds(c * _SOFF, 512)],
                            regr_hbm.at[c, b, pl.ds(5120, 512)])
        pltpu.sync_copy(lstage.at[pl.ds(0, 512)],
                        lab_hbm.at[0, b, pl.ds(5120, 512)])


@jax.jit
def _sc_assign(gt_boxes, msw):
    mesh = plsc.VectorSubcoreMesh(core_axis_name="c", subcore_axis_name="s")
    f = pl.kernel(
        _sc_body,
        out_type=(
            jax.ShapeDtypeStruct((6, _B, _NPIXP), _F32),
            jax.ShapeDtypeStruct((1, _B, _NPIXP), _I32),
        ),
        mesh=mesh,
        compiler_params=pltpu.CompilerParams(needs_layout_passes=False),
        scratch_types=[
            pltpu.VMEM((_N * 5,), _F32),
            pltpu.VMEM((_N * 5,), _F32),
            pltpu.VMEM((112,), _I32),
            pltpu.VMEM((112,), _I32),
            pltpu.VMEM((112,), _I32),
            pltpu.VMEM((112,), _I32),
            pltpu.VMEM((112,), _I32),
            pltpu.VMEM((112,), _F32),
            pltpu.VMEM((112,), _F32),
            pltpu.VMEM((112,), _F32),
            pltpu.VMEM((112,), _F32),
            pltpu.VMEM((2064,), _F32),
            pltpu.VMEM((2064,), _I32),
            pltpu.VMEM((_SOFF * 6 + 16,), _F32),
            pltpu.VMEM((2064,), _I32),
        ],
    )
    return f(gt_boxes.reshape(_B, _N * 5), msw.reshape(_B, _N * 5))


def _tc_finish_body(lab_ref, sm_ref, cls_ref, regr_ref):
    labi = lab_ref[0, :, :_NPIX]
    for c in range(_NUM_CLASSES):
        cls_ref[c] = (labi == c).astype(_F32)
    cls_ref[_NUM_CLASSES] = sm_ref[4, :, :_NPIX]
    cls_ref[_NUM_CLASSES + 1] = sm_ref[5, :, :_NPIX]
    regr_ref[...] = sm_ref[:, :, :_NPIX]


@jax.jit
def _tc_finish(lab, regr_soa):
    nc2 = _NUM_CLASSES + 2
    return pl.pallas_call(
        _tc_finish_body,
        out_shape=[
            jax.ShapeDtypeStruct((nc2, _B, _NPIX), _F32),
            jax.ShapeDtypeStruct((6, _B, _NPIX), _F32),
        ],
        compiler_params=pltpu.CompilerParams(
            vmem_limit_bytes=100 * 1024 * 1024),
    )(lab, regr_soa)


def kernel(fm_shapes, gt_boxes, meta_select_weight):
    del fm_shapes
    regr_soa, lab = _sc_assign(gt_boxes, meta_select_weight)
    cls_p, regr_p = _tc_finish(lab, regr_soa)
    cls_t = jnp.transpose(cls_p, (1, 2, 0))
    regr_t = jnp.transpose(regr_p, (1, 2, 0))
    return cls_t, regr_t


# speedup vs baseline: 3.6926x; 1.0062x over previous
import jax
import jax.numpy as jnp
from jax import lax
from jax.experimental import pallas as pl
from jax.experimental.pallas import tpu as pltpu
from jax.experimental.pallas import tpu_sc as plsc

_NUM_CLASSES = 80
_SHRINK = 0.2
_B = 8
_N = 100
_NPIX = 5456
_NPIXP = 5632
_F32 = jnp.float32
_I32 = jnp.int32
_SOFF = 2048


def _unit_body(lvl, b, y0, nrows, out_base,
               gt_hbm, msw_hbm,
               boxes_v, msw_v, px1_v, py1_v, px2_v, py2_v, val_v,
               bx1_v, by1_v, bx2_v, by2_v, besta, besti, rstage, lstage):
    iot = lax.iota(_I32, 16)
    log2fw = 6 - lvl
    fw = lax.shift_left(1, log2fw)
    fh = fw
    stride = lax.shift_left(8, lvl)
    stride_f = stride.astype(_F32)
    inv_s = jnp.full((16,), 1.0, _F32) / lax.broadcast(stride_f, (16,))
    inv4s = inv_s * _F32(0.25)
    npx = nrows * fw

    def setup_body(g, _):
        ridx = g * 16 + iot
        gmask = ridx < _N
        fidx = ridx * 5
        x1 = plsc.load_gather(boxes_v, [fidx], mask=gmask)
        y1 = plsc.load_gather(boxes_v, [fidx + 1], mask=gmask)
        x2 = plsc.load_gather(boxes_v, [fidx + 2], mask=gmask)
        y2 = plsc.load_gather(boxes_v, [fidx + 3], mask=gmask)
        valid = ((jnp.abs(x1) + jnp.abs(y1) + jnp.abs(x2) + jnp.abs(y2))
                 > 0.0) & gmask
        cx = (x1 + x2) * _F32(0.5)
        cy = (y1 + y2) * _F32(0.5)
        w = x2 - x1
        h = y2 - y1
        sx1 = (cx - w * _F32(_SHRINK) * _F32(0.5)) * inv_s
        sy1 = (cy - h * _F32(_SHRINK) * _F32(0.5)) * inv_s
        sx2 = (cx + w * _F32(_SHRINK) * _F32(0.5)) * inv_s
        sy2 = (cy + h * _F32(_SHRINK) * _F32(0.5)) * inv_s
        p1 = jnp.clip(sx1.astype(_I32), 0, fw - 1)
        q1 = jnp.clip(sy1.astype(_I32), 0, fh - 1)
        c2 = sx2.astype(_I32)
        c2 = c2 + jnp.where(c2.astype(_F32) < sx2, 1, 0)
        r2 = sy2.astype(_I32)
        r2 = r2 + jnp.where(r2.astype(_F32) < sy2, 1, 0)
        p2 = jnp.clip(c2, p1 + 1, fw)
        q2 = jnp.clip(r2, q1 + 1, fh)
        sl = pl.ds(g * 16, 16)
        px1_v[sl] = p1
        py1_v[sl] = q1
        px2_v[sl] = p2
        py2_v[sl] = q2
        val_v[sl] = jnp.where(valid, 1, 0)
        bx1_v[sl] = x1
        by1_v[sl] = y1
        bx2_v[sl] = x2
        by2_v[sl] = y2
        return 0

    lax.fori_loop(0, 7, setup_body, 0)

    big = jnp.full((16,), 1e7, dtype=_F32)
    zer = jnp.zeros((16,), dtype=_I32)

    def init_body(g, _):
        besta[pl.ds(g * 16, 16)] = big
        besti[pl.ds(g * 16, 16)] = zer
        return 0

    lax.fori_loop(0, lax.shift_right_logical(npx, 4), init_body, 0)

    def box_grp_body(g, _):
        gs = pl.ds(g * 16, 16)
        p1v = px1_v[gs]
        p2v = px2_v[gs]
        q1v = py1_v[gs]
        q2v = py2_v[gs]
        vv = val_v[gs]
        x1v = bx1_v[gs]
        y1v = by1_v[gs]
        x2v = bx2_v[gs]
        y2v = by2_v[gs]
        for j in range(16):
            n = g * 16 + j
            v = vv[j]
            p1 = p1v[j]
            q1 = q1v[j]
            q2 = q2v[j]
            x1f = x1v[j]
            y1f = y1v[j]
            x2f = x2v[j]
            y2f = y2v[j]
            ry1 = jnp.maximum(q1, y0)
            ry2 = jnp.minimum(q2, y0 + nrows)
            ry2 = jnp.where(v > 0, jnp.maximum(ry2, ry1), ry1)
            rw = p2v[j] - p1
            m_in = iot < rw
            xsf = (p1 + iot).astype(_F32)
            sx = (xsf + _F32(0.5)) * stride_f
            dl = jnp.maximum(sx - x1f, 0.0)
            dr = jnp.maximum(x2f - sx, 0.0)
            dlr = dl + dr
            nvec = lax.broadcast(n, (16,))

            def row_body(y, _, dlr=dlr, m_in=m_in, y1f=y1f, y2f=y2f,
                         p1=p1, nvec=nvec):
                sy = (y.astype(_F32) + _F32(0.5)) * stride_f
                dt = jnp.maximum(sy - y1f, 0.0)
                db = jnp.maximum(y2f - sy, 0.0)
                area = dlr * (dt + db)
                loc = (y - y0) * fw + p1
                sl = pl.ds(loc, 16)
                cur = besta[sl]
                upd = m_in & (area < cur)
                besta[sl] = jnp.where(upd, area, cur)
                curi = besti[sl]
                besti[sl] = jnp.where(upd, nvec, curi)
                return 0

            lax.fori_loop(ry1, ry2, row_body, 0)
        return 0

    lax.fori_loop(0, 7, box_grp_body, 0)

    def grp_body(g, _):
        base = g * 16
        lp = base + iot
        sl = pl.ds(base, 16)
        idxv = besti[sl]
        areav = besta[sl]
        pos = areav < 1e7
        posf = jnp.where(pos, _F32(1.0), _F32(0.0))
        x = lp & (fw - 1)
        y = y0 + lax.shift_right_logical(lp, log2fw)
        sx = (x.astype(_F32) + _F32(0.5)) * stride_f
        sy = (y.astype(_F32) + _F32(0.5)) * stride_f
        idx5 = idxv * 5
        bx1 = plsc.load_gather(boxes_v, [idx5])
        by1 = plsc.load_gather(boxes_v, [idx5 + 1])
        bx2 = plsc.load_gather(boxes_v, [idx5 + 2])
        by2 = plsc.load_gather(boxes_v, [idx5 + 3])
        labf = plsc.load_gather(boxes_v, [idx5 + 4])
        mw = plsc.load_gather(msw_v, [idx5 + lvl])
        dl = jnp.maximum(sx - bx1, 0.0)
        dt = jnp.maximum(sy - by1, 0.0)
        dr = jnp.maximum(bx2 - sx, 0.0)
        db = jnp.maximum(by2 - sy, 0.0)
        apn = jnp.minimum(dl, dr) * jnp.minimum(dt, db)
        apd = jnp.maximum(jnp.maximum(dl, dr) * jnp.maximum(dt, db), 1e-12)
        soft = jnp.where(pos, (apn / apd) * mw, _F32(1.0))
        ob = out_base + base
        rstage[pl.ds(0 * _SOFF + ob, 16)] = dl * inv4s * posf
        rstage[pl.ds(1 * _SOFF + ob, 16)] = dt * inv4s * posf
        rstage[pl.ds(2 * _SOFF + ob, 16)] = dr * inv4s * posf
        rstage[pl.ds(3 * _SOFF + ob, 16)] = db * inv4s * posf
        rstage[pl.ds(4 * _SOFF + ob, 16)] = soft
        rstage[pl.ds(5 * _SOFF + ob, 16)] = posf
        labo = jnp.where(pos, labf.astype(_I32), -1)
        lstage[pl.ds(ob, 16)] = labo
        return 0

    lax.fori_loop(0, lax.shift_right_logical(npx, 4), grp_body, 0)


def _sc_body(gt_hbm, msw_hbm, regr_hbm, lab_hbm, boxes_v, msw_v,
             px1_v, py1_v, px2_v, py2_v, val_v,
             bx1_v, by1_v, bx2_v, by2_v, besta, besti, rstage, lstage, sem):
    cid = lax.axis_index("c")
    sid = lax.axis_index("s")
    wid = sid * 2 + cid
    scr = (boxes_v, msw_v, px1_v, py1_v, px2_v, py2_v, val_v,
           bx1_v, by1_v, bx2_v, by2_v, besta, besti, rstage, lstage)

    is_a = wid < 16
    is_b = (wid >= 16) & (wid < 24)
    b = jnp.where(is_a, lax.div(wid, 2),
                  jnp.where(is_b, wid - 16, wid - 24))
    nunits = jnp.where(wid < 24, 1, 3)

    pltpu.sync_copy(gt_hbm.at[b], boxes_v)
    pltpu.sync_copy(msw_hbm.at[b], msw_v)

    def unit(u, _):
        lvl = jnp.where(is_a, 0, jnp.where(is_b, 1, 2 + u))
        y0 = jnp.where(is_a, (wid & 1) * 32, 0)
        fh = lax.shift_left(1, 6 - lvl)
        nrows = jnp.where(is_a, 32, fh)
        out_base = jnp.where(u == 1, 256, jnp.where(u == 2, 320, 0))
        _unit_body(lvl, b, y0, nrows, out_base, gt_hbm, msw_hbm, *scr)
        return 0

    lax.fori_loop(0, nunits, unit, 0)

    def _flush(p0, n):
        copies = []
        for c in range(6):
            copies.append(pltpu.make_async_copy(
                rstage.at[pl.ds(c * _SOFF, n)],
                regr_hbm.at[c, b, pl.ds(p0, n)], sem))
        copies.append(pltpu.make_async_copy(
            lstage.at[pl.ds(0, n)], lab_hbm.at[0, b, pl.ds(p0, n)], sem))
        for cp in copies:
            cp.start()
        for cp in copies:
            cp.wait()

    @pl.when(is_a)
    def _():
        _flush((wid & 1) * 2048, 2048)

    @pl.when(is_b)
    def _():
        _flush(4096, 1024)

    @pl.when(wid >= 24)
    def _():
        _flush(5120, 512)


@jax.jit
def _sc_assign(gt_boxes, msw):
    mesh = plsc.VectorSubcoreMesh(core_axis_name="c", subcore_axis_name="s")
    f = pl.kernel(
        _sc_body,
        out_type=(
            jax.ShapeDtypeStruct((6, _B, _NPIXP), _F32),
            jax.ShapeDtypeStruct((1, _B, _NPIXP), _I32),
        ),
        mesh=mesh,
        compiler_params=pltpu.CompilerParams(needs_layout_passes=False),
        scratch_types=[
            pltpu.VMEM((_N * 5,), _F32),
            pltpu.VMEM((_N * 5,), _F32),
            pltpu.VMEM((112,), _I32),
            pltpu.VMEM((112,), _I32),
            pltpu.VMEM((112,), _I32),
            pltpu.VMEM((112,), _I32),
            pltpu.VMEM((112,), _I32),
            pltpu.VMEM((112,), _F32),
            pltpu.VMEM((112,), _F32),
            pltpu.VMEM((112,), _F32),
            pltpu.VMEM((112,), _F32),
            pltpu.VMEM((2064,), _F32),
            pltpu.VMEM((2064,), _I32),
            pltpu.VMEM((_SOFF * 6 + 16,), _F32),
            pltpu.VMEM((2064,), _I32),
            pltpu.SemaphoreType.DMA,
        ],
    )
    return f(gt_boxes.reshape(_B, _N * 5), msw.reshape(_B, _N * 5))


def _tc_finish_body(lab_ref, sm_ref, cls_ref, regr_ref):
    labi = lab_ref[0, :, :_NPIX]
    for c in range(_NUM_CLASSES):
        cls_ref[c] = (labi == c).astype(_F32)
    cls_ref[_NUM_CLASSES] = sm_ref[4, :, :_NPIX]
    cls_ref[_NUM_CLASSES + 1] = sm_ref[5, :, :_NPIX]
    regr_ref[...] = sm_ref[:, :, :_NPIX]


@jax.jit
def _tc_finish(lab, regr_soa):
    nc2 = _NUM_CLASSES + 2
    return pl.pallas_call(
        _tc_finish_body,
        out_shape=[
            jax.ShapeDtypeStruct((nc2, _B, _NPIX), _F32),
            jax.ShapeDtypeStruct((6, _B, _NPIX), _F32),
        ],
        compiler_params=pltpu.CompilerParams(
            vmem_limit_bytes=100 * 1024 * 1024),
    )(lab, regr_soa)


def kernel(fm_shapes, gt_boxes, meta_select_weight):
    del fm_shapes
    regr_soa, lab = _sc_assign(gt_boxes, meta_select_weight)
    cls_p, regr_p = _tc_finish(lab, regr_soa)
    cls_t = jnp.transpose(cls_p, (1, 2, 0))
    regr_t = jnp.transpose(regr_p, (1, 2, 0))
    return cls_t, regr_t
